# Initial kernel scaffold; baseline (speedup 1.0000x reference)
import jax
import jax.numpy as jnp
from jax.experimental import pallas as pl


def kernel(x, edge_index1, edge_index2, W_l0, b_l0, W_r0, W_l1, b_l1, W_r1):
    raise NotImplementedError("write your pallas kernel here")


# speedup vs baseline: 3.2390x; 3.2390x over previous
import functools

import jax
import jax.numpy as jnp
from jax import lax
from jax.experimental import pallas as pl
from jax.experimental.pallas import tpu as pltpu
from jax.experimental.pallas import tpu_sc as plsc

N = 10000
D = 128
E = 320000
NP = 10240
EP = 327680
CH = 128
NCH = EP // (32 * CH)
ROWS_PER_TILE = NP // 16


def _sc_aggregate(x_pad, src_r, dst_r, z_feat, z_cnt):
    mesh = plsc.VectorSubcoreMesh(
        core_axis_name="c", subcore_axis_name="s", num_cores=2, num_subcores=16
    )

    @functools.partial(
        pl.kernel,
        out_type=[
            jax.ShapeDtypeStruct((2, NP, D), jnp.float32),
            jax.ShapeDtypeStruct((32, NP), jnp.float32),
        ],
        mesh=mesh,
        compiler_params=pltpu.CompilerParams(needs_layout_passes=False),
        scratch_types=[
            pltpu.MemorySpace.VMEM_SHARED((NP, D), jnp.float32),
            pltpu.VMEM((NP,), jnp.float32),
            pltpu.VMEM((CH,), jnp.int32),
            pltpu.VMEM((CH,), jnp.int32),
            pltpu.VMEM((CH,), jnp.int32),
            pltpu.VMEM((CH,), jnp.int32),
            pltpu.VMEM((CH, D), jnp.float32),
            pltpu.VMEM((CH, D), jnp.float32),
            pltpu.SemaphoreType.DMA,
            pltpu.SemaphoreType.DMA,
        ],
    )
    def agg(x_hbm, src_hbm, dst_hbm, zf_hbm, zc_hbm, feat_out, cnt_out,
            acc_s, cnt_v, sidx0, sidx1, didx0, didx1, rows0, rows1,
            sem0, sem1):
        cid = lax.axis_index("c")
        sid = lax.axis_index("s")
        tile = cid * 16 + sid
        r0 = sid * ROWS_PER_TILE
        ones = jnp.ones((16,), jnp.float32)

        pltpu.sync_copy(zc_hbm, cnt_v)
        pltpu.sync_copy(zf_hbm, rows0)

        @pl.loop(0, ROWS_PER_TILE // CH)
        def _zero(j):
            rj = pl.multiple_of(r0 + j * CH, CH)
            pltpu.sync_copy(rows0, acc_s.at[pl.ds(rj, CH)])

        plsc.subcore_barrier()

        @pl.loop(0, NCH, step=2)
        def _chunks(k):
            crow = tile * NCH + k
            pltpu.sync_copy(src_hbm.at[crow], sidx0)
            pltpu.sync_copy(src_hbm.at[crow + 1], sidx1)
            pltpu.sync_copy(dst_hbm.at[crow], didx0)
            pltpu.sync_copy(dst_hbm.at[crow + 1], didx1)
            c0 = pltpu.async_copy(x_hbm.at[sidx0], rows0, sem0)
            c1 = pltpu.async_copy(x_hbm.at[sidx1], rows1, sem1)
            for j in range(CH // 16):
                plsc.addupdate_scatter(cnt_v, [didx0[pl.ds(j * 16, 16)]], ones)
                plsc.addupdate_scatter(cnt_v, [didx1[pl.ds(j * 16, 16)]], ones)
            c0.wait()
            pltpu.sync_copy(rows0, acc_s.at[didx0], add=True)
            c1.wait()
            pltpu.sync_copy(rows1, acc_s.at[didx1], add=True)

        plsc.subcore_barrier()

        @pl.loop(0, ROWS_PER_TILE // CH)
        def _dump(j):
            rj = pl.multiple_of(r0 + j * CH, CH)
            pltpu.sync_copy(acc_s.at[pl.ds(rj, CH)], rows0)
            pltpu.sync_copy(rows0, feat_out.at[cid, pl.ds(rj, CH)])

        pltpu.sync_copy(cnt_v, cnt_out.at[tile])

    return agg(x_pad, src_r, dst_r, z_feat, z_cnt)


def _tc_combine(acc, cnt_t, x_pad, W_l, b_l, W_r, relu):
    BR = 1024
    NB = NP // BR

    def body(acc_ref, cnt_ref, x_ref, wl_ref, bl_ref, wr_ref, o_ref):
        a = acc_ref[...]
        s = a[0] + a[1]
        cn = jnp.sum(cnt_ref[...], axis=1, keepdims=True)
        mean = s * (1.0 / jnp.maximum(cn, 1.0))
        h = (jnp.dot(mean, wl_ref[...], preferred_element_type=jnp.float32)
             + bl_ref[...]
             + jnp.dot(x_ref[...], wr_ref[...], preferred_element_type=jnp.float32))
        if relu:
            h = jnp.maximum(h, 0.0)
        o_ref[...] = h

    return pl.pallas_call(
        body,
        grid=(NB,),
        in_specs=[
            pl.BlockSpec((2, BR, D), lambda i: (0, i, 0)),
            pl.BlockSpec((BR, 32), lambda i: (i, 0)),
            pl.BlockSpec((BR, D), lambda i: (i, 0)),
            pl.BlockSpec((D, D), lambda i: (0, 0)),
            pl.BlockSpec((1, D), lambda i: (0, 0)),
            pl.BlockSpec((D, D), lambda i: (0, 0)),
        ],
        out_specs=pl.BlockSpec((BR, D), lambda i: (i, 0)),
        out_shape=jax.ShapeDtypeStruct((NP, D), jnp.float32),
    )(acc, cnt_t, x_pad, W_l, b_l, W_r)


def _pad_edges(edge_index):
    src = jnp.concatenate(
        [edge_index[0].astype(jnp.int32), jnp.zeros((EP - E,), jnp.int32)]
    ).reshape(EP // CH, CH)
    dst = jnp.concatenate(
        [edge_index[1].astype(jnp.int32), jnp.full((EP - E,), N, jnp.int32)]
    ).reshape(EP // CH, CH)
    return src, dst


def kernel(x, edge_index1, edge_index2, W_l0, b_l0, W_r0, W_l1, b_l1, W_r1):
    src1, dst1 = _pad_edges(edge_index1)
    src2, dst2 = _pad_edges(edge_index2)
    z_feat = jnp.zeros((CH, D), jnp.float32)
    z_cnt = jnp.zeros((NP,), jnp.float32)
    x_pad = jnp.zeros((NP, D), jnp.float32).at[:N].set(x)

    acc1, cnt1 = _sc_aggregate(x_pad, src1, dst1, z_feat, z_cnt)
    h_pad = _tc_combine(acc1, cnt1.T, x_pad, W_l0, b_l0.reshape(1, D), W_r0,
                        True)

    acc2, cnt2 = _sc_aggregate(h_pad, src2, dst2, z_feat, z_cnt)
    out = _tc_combine(acc2, cnt2.T, h_pad, W_l1, b_l1.reshape(1, D), W_r1,
                      False)
    return out[:N]


# speedup vs baseline: 3.5353x; 1.0915x over previous
import functools

import jax
import jax.numpy as jnp
from jax import lax
from jax.experimental import pallas as pl
from jax.experimental.pallas import tpu as pltpu
from jax.experimental.pallas import tpu_sc as plsc

N = 10000
D = 128
E = 320000
NP = 10240
EP = 327680
CH = 128
NCH = EP // (32 * CH)
ROWS_PER_TILE = NP // 16


def _sc_aggregate(x_pad, src_r, dst_r, z_feat, z_cnt):
    mesh = plsc.VectorSubcoreMesh(
        core_axis_name="c", subcore_axis_name="s", num_cores=2, num_subcores=16
    )

    @functools.partial(
        pl.kernel,
        out_type=[
            jax.ShapeDtypeStruct((2, NP, D), jnp.float32),
            jax.ShapeDtypeStruct((32, NP), jnp.float32),
        ],
        mesh=mesh,
        compiler_params=pltpu.CompilerParams(needs_layout_passes=False),
        scratch_types=[
            pltpu.MemorySpace.VMEM_SHARED((NP, D), jnp.float32),
            pltpu.VMEM((NP,), jnp.float32),
            pltpu.VMEM((CH,), jnp.int32),
            pltpu.VMEM((CH,), jnp.int32),
            pltpu.VMEM((CH,), jnp.int32),
            pltpu.VMEM((CH,), jnp.int32),
            pltpu.VMEM((CH, D), jnp.float32),
            pltpu.VMEM((CH, D), jnp.float32),
            pltpu.SemaphoreType.DMA,
            pltpu.SemaphoreType.DMA,
            pltpu.SemaphoreType.DMA,
            pltpu.SemaphoreType.DMA,
            pltpu.SemaphoreType.DMA,
        ],
    )
    def agg(x_hbm, src_hbm, dst_hbm, zf_hbm, zc_hbm, feat_out, cnt_out,
            acc_s, cnt_v, sidx0, sidx1, didx0, didx1, rows0, rows1,
            semi, semg0, semg1, sems0, sems1):
        cid = lax.axis_index("c")
        sid = lax.axis_index("s")
        tile = cid * 16 + sid
        r0 = sid * ROWS_PER_TILE
        ones = jnp.ones((16,), jnp.float32)

        pltpu.sync_copy(zc_hbm, cnt_v)
        pltpu.sync_copy(zf_hbm, rows0)

        @pl.loop(0, ROWS_PER_TILE // CH)
        def _zero(j):
            rj = pl.multiple_of(r0 + j * CH, CH)
            pltpu.sync_copy(rows0, acc_s.at[pl.ds(rj, CH)])

        plsc.subcore_barrier()

        @pl.loop(0, NCH, step=2)
        def _chunks(k):
            @pl.when(k > 0)
            def _():
                pltpu.make_async_copy(rows0, acc_s.at[didx0], sems0).wait()
                pltpu.make_async_copy(rows1, acc_s.at[didx1], sems1).wait()

            crow = tile * NCH + k
            i0 = pltpu.async_copy(src_hbm.at[crow], sidx0, semi)
            i1 = pltpu.async_copy(src_hbm.at[crow + 1], sidx1, semi)
            i2 = pltpu.async_copy(dst_hbm.at[crow], didx0, semi)
            i3 = pltpu.async_copy(dst_hbm.at[crow + 1], didx1, semi)
            i0.wait(); i1.wait(); i2.wait(); i3.wait()
            g0 = pltpu.async_copy(x_hbm.at[sidx0], rows0, semg0)
            g1 = pltpu.async_copy(x_hbm.at[sidx1], rows1, semg1)
            for j in range(CH // 16):
                plsc.addupdate_scatter(cnt_v, [didx0[pl.ds(j * 16, 16)]], ones)
                plsc.addupdate_scatter(cnt_v, [didx1[pl.ds(j * 16, 16)]], ones)
            g0.wait()
            pltpu.async_copy(rows0, acc_s.at[didx0], sems0, add=True)
            g1.wait()
            pltpu.async_copy(rows1, acc_s.at[didx1], sems1, add=True)

        pltpu.make_async_copy(rows0, acc_s.at[didx0], sems0).wait()
        pltpu.make_async_copy(rows1, acc_s.at[didx1], sems1).wait()
        plsc.subcore_barrier()

        @pl.loop(0, ROWS_PER_TILE // CH)
        def _dump(j):
            rj = pl.multiple_of(r0 + j * CH, CH)
            pltpu.sync_copy(acc_s.at[pl.ds(rj, CH)], rows0)
            pltpu.sync_copy(rows0, feat_out.at[cid, pl.ds(rj, CH)])

        pltpu.sync_copy(cnt_v, cnt_out.at[tile])

    return agg(x_pad, src_r, dst_r, z_feat, z_cnt)


def _tc_combine(acc, cnt_t, x_pad, W_l, b_l, W_r, relu):
    BR = 1024
    NB = NP // BR

    def body(acc_ref, cnt_ref, x_ref, wl_ref, bl_ref, wr_ref, o_ref):
        a = acc_ref[...]
        s = a[0] + a[1]
        cn = jnp.sum(cnt_ref[...], axis=1, keepdims=True)
        mean = s * (1.0 / jnp.maximum(cn, 1.0))
        h = (jnp.dot(mean, wl_ref[...], preferred_element_type=jnp.float32)
             + bl_ref[...]
             + jnp.dot(x_ref[...], wr_ref[...], preferred_element_type=jnp.float32))
        if relu:
            h = jnp.maximum(h, 0.0)
        o_ref[...] = h

    return pl.pallas_call(
        body,
        grid=(NB,),
        in_specs=[
            pl.BlockSpec((2, BR, D), lambda i: (0, i, 0)),
            pl.BlockSpec((BR, 32), lambda i: (i, 0)),
            pl.BlockSpec((BR, D), lambda i: (i, 0)),
            pl.BlockSpec((D, D), lambda i: (0, 0)),
            pl.BlockSpec((1, D), lambda i: (0, 0)),
            pl.BlockSpec((D, D), lambda i: (0, 0)),
        ],
        out_specs=pl.BlockSpec((BR, D), lambda i: (i, 0)),
        out_shape=jax.ShapeDtypeStruct((NP, D), jnp.float32),
    )(acc, cnt_t, x_pad, W_l, b_l, W_r)


def _pad_edges(edge_index):
    src = jnp.concatenate(
        [edge_index[0].astype(jnp.int32), jnp.zeros((EP - E,), jnp.int32)]
    ).reshape(EP // CH, CH)
    dst = jnp.concatenate(
        [edge_index[1].astype(jnp.int32), jnp.full((EP - E,), N, jnp.int32)]
    ).reshape(EP // CH, CH)
    return src, dst


def kernel(x, edge_index1, edge_index2, W_l0, b_l0, W_r0, W_l1, b_l1, W_r1):
    src1, dst1 = _pad_edges(edge_index1)
    src2, dst2 = _pad_edges(edge_index2)
    z_feat = jnp.zeros((CH, D), jnp.float32)
    z_cnt = jnp.zeros((NP,), jnp.float32)
    x_pad = jnp.zeros((NP, D), jnp.float32).at[:N].set(x)

    acc1, cnt1 = _sc_aggregate(x_pad, src1, dst1, z_feat, z_cnt)
    h_pad = _tc_combine(acc1, cnt1.T, x_pad, W_l0, b_l0.reshape(1, D), W_r0,
                        True)

    acc2, cnt2 = _sc_aggregate(h_pad, src2, dst2, z_feat, z_cnt)
    out = _tc_combine(acc2, cnt2.T, h_pad, W_l1, b_l1.reshape(1, D), W_r1,
                      False)
    return out[:N]


# speedup vs baseline: 3.5594x; 1.0068x over previous
import functools

import jax
import jax.numpy as jnp
from jax import lax
from jax.experimental import pallas as pl
from jax.experimental.pallas import tpu as pltpu
from jax.experimental.pallas import tpu_sc as plsc

N = 10000
D = 128
E = 320000
NP = 10240
EP = 327680
CH = 128
NCH = EP // (32 * CH)
ROWS_PER_TILE = NP // 16


def _sc_aggregate(x_in, src_r, dst_r, z_feat, z_cnt):
    mesh = plsc.VectorSubcoreMesh(
        core_axis_name="c", subcore_axis_name="s", num_cores=2, num_subcores=16
    )

    @functools.partial(
        pl.kernel,
        out_type=[
            jax.ShapeDtypeStruct((2, NP, D), jnp.float32),
            jax.ShapeDtypeStruct((32, NP), jnp.float32),
        ],
        mesh=mesh,
        compiler_params=pltpu.CompilerParams(needs_layout_passes=False),
        scratch_types=[
            pltpu.MemorySpace.VMEM_SHARED((NP, D), jnp.float32),
            pltpu.VMEM((NP,), jnp.float32),
            pltpu.VMEM((CH,), jnp.int32),
            pltpu.VMEM((CH,), jnp.int32),
            pltpu.VMEM((CH,), jnp.int32),
            pltpu.VMEM((CH,), jnp.int32),
            pltpu.VMEM((CH, D), jnp.float32),
            pltpu.VMEM((CH, D), jnp.float32),
            pltpu.SemaphoreType.DMA,
            pltpu.SemaphoreType.DMA,
            pltpu.SemaphoreType.DMA,
            pltpu.SemaphoreType.DMA,
            pltpu.SemaphoreType.DMA,
        ],
    )
    def agg(x_hbm, src_hbm, dst_hbm, zf_hbm, zc_hbm, feat_out, cnt_out,
            acc_s, cnt_v, sidx0, sidx1, didx0, didx1, rows0, rows1,
            semi, semg0, semg1, sems0, sems1):
        cid = lax.axis_index("c")
        sid = lax.axis_index("s")
        tile = cid * 16 + sid
        r0 = sid * ROWS_PER_TILE
        ones = jnp.ones((16,), jnp.float32)

        pltpu.sync_copy(zc_hbm, cnt_v)
        pltpu.sync_copy(zf_hbm, rows0)

        zws = []
        for j in range(ROWS_PER_TILE // CH):
            rj = pl.multiple_of(r0 + j * CH, CH)
            zws.append(pltpu.async_copy(rows0, acc_s.at[pl.ds(rj, CH)], semg0))
        for w in zws:
            w.wait()
        plsc.subcore_barrier()

        @pl.loop(0, NCH, step=2)
        def _chunks(k):
            @pl.when(k > 0)
            def _():
                pltpu.make_async_copy(rows0, acc_s.at[didx0], sems0).wait()
                pltpu.make_async_copy(rows1, acc_s.at[didx1], sems1).wait()

            crow = tile * NCH + k
            i0 = pltpu.async_copy(src_hbm.at[crow], sidx0, semi)
            i1 = pltpu.async_copy(src_hbm.at[crow + 1], sidx1, semi)
            i2 = pltpu.async_copy(dst_hbm.at[crow], didx0, semi)
            i3 = pltpu.async_copy(dst_hbm.at[crow + 1], didx1, semi)
            i0.wait(); i1.wait(); i2.wait(); i3.wait()
            g0 = pltpu.async_copy(x_hbm.at[sidx0], rows0, semg0)
            g1 = pltpu.async_copy(x_hbm.at[sidx1], rows1, semg1)
            for j in range(CH // 16):
                plsc.addupdate_scatter(cnt_v, [didx0[pl.ds(j * 16, 16)]], ones)
                plsc.addupdate_scatter(cnt_v, [didx1[pl.ds(j * 16, 16)]], ones)
            g0.wait()
            pltpu.async_copy(rows0, acc_s.at[didx0], sems0, add=True)
            g1.wait()
            pltpu.async_copy(rows1, acc_s.at[didx1], sems1, add=True)

        pltpu.make_async_copy(rows0, acc_s.at[didx0], sems0).wait()
        pltpu.make_async_copy(rows1, acc_s.at[didx1], sems1).wait()
        plsc.subcore_barrier()

        cw = pltpu.async_copy(cnt_v, cnt_out.at[tile], semi)
        rbufs = (rows0, rows1)
        sgs = (semg0, semg1)
        sss = (sems0, sems1)
        nblk = ROWS_PER_TILE // CH
        for j in range(nblk):
            rj = pl.multiple_of(r0 + j * CH, CH)
            b = j % 2
            if j >= 2:
                pltpu.make_async_copy(rbufs[b], feat_out.at[cid, pl.ds(rj, CH)],
                                      sss[b]).wait()
            pltpu.async_copy(acc_s.at[pl.ds(rj, CH)], rbufs[b], sgs[b]).wait()
            pltpu.async_copy(rbufs[b], feat_out.at[cid, pl.ds(rj, CH)], sss[b])
        for j in range(max(nblk - 2, 0), nblk):
            rj = pl.multiple_of(r0 + j * CH, CH)
            pltpu.make_async_copy(rbufs[j % 2], feat_out.at[cid, pl.ds(rj, CH)],
                                  sss[j % 2]).wait()
        cw.wait()

    return agg(x_in, src_r, dst_r, z_feat, z_cnt)


def _tc_combine(acc, cnt_t, x_in, W_l, b_l, W_r, relu):
    BR = 1000
    NB = N // BR

    def body(acc_ref, cnt_ref, x_ref, wl_ref, bl_ref, wr_ref, o_ref):
        a = acc_ref[...]
        s = a[0] + a[1]
        cn = jnp.sum(cnt_ref[...], axis=1, keepdims=True)
        mean = s * (1.0 / jnp.maximum(cn, 1.0))
        h = (jnp.dot(mean, wl_ref[...], preferred_element_type=jnp.float32)
             + bl_ref[...]
             + jnp.dot(x_ref[...], wr_ref[...], preferred_element_type=jnp.float32))
        if relu:
            h = jnp.maximum(h, 0.0)
        o_ref[...] = h

    return pl.pallas_call(
        body,
        grid=(NB,),
        in_specs=[
            pl.BlockSpec((2, BR, D), lambda i: (0, i, 0)),
            pl.BlockSpec((BR, 32), lambda i: (i, 0)),
            pl.BlockSpec((BR, D), lambda i: (i, 0)),
            pl.BlockSpec((D, D), lambda i: (0, 0)),
            pl.BlockSpec((1, D), lambda i: (0, 0)),
            pl.BlockSpec((D, D), lambda i: (0, 0)),
        ],
        out_specs=pl.BlockSpec((BR, D), lambda i: (i, 0)),
        out_shape=jax.ShapeDtypeStruct((N, D), jnp.float32),
    )(acc, cnt_t, x_in, W_l, b_l, W_r)


def _pad_edges(edge_index):
    src = jnp.concatenate(
        [edge_index[0].astype(jnp.int32), jnp.zeros((EP - E,), jnp.int32)]
    ).reshape(EP // CH, CH)
    dst = jnp.concatenate(
        [edge_index[1].astype(jnp.int32), jnp.full((EP - E,), N, jnp.int32)]
    ).reshape(EP // CH, CH)
    return src, dst


def kernel(x, edge_index1, edge_index2, W_l0, b_l0, W_r0, W_l1, b_l1, W_r1):
    src1, dst1 = _pad_edges(edge_index1)
    src2, dst2 = _pad_edges(edge_index2)
    z_feat = jnp.zeros((CH, D), jnp.float32)
    z_cnt = jnp.zeros((NP,), jnp.float32)

    acc1, cnt1 = _sc_aggregate(x, src1, dst1, z_feat, z_cnt)
    h = _tc_combine(acc1, cnt1.T, x, W_l0, b_l0.reshape(1, D), W_r0, True)

    acc2, cnt2 = _sc_aggregate(h, src2, dst2, z_feat, z_cnt)
    return _tc_combine(acc2, cnt2.T, h, W_l1, b_l1.reshape(1, D), W_r1, False)
